# baseline (speedup 1.0000x reference)
import functools

import jax
import jax.numpy as jnp
from jax import lax
from jax.experimental import pallas as pl
from jax.experimental.pallas import tpu as pltpu
from jax.experimental.pallas import tpu_sc as plsc

N = 10000
E = 320000
D_IN = 128
H = 146
NUM_LAYERS = 4
B = 8
C = 10
EPS = 1e-5

HP = 192
HH = HP // 2
NC = 2
NS = 16
K = 128
NBUF = 2
CH = NBUF * (-(-E // (NS * K * NBUF)))
EPAD = NS * CH * K
NP = N + 112
RPT = NP // NS
CHD = -(-CH // 2)

_sc_mesh = plsc.VectorSubcoreMesh(
    core_axis_name="c", subcore_axis_name="s", num_cores=NC, num_subcores=NS)
_sc_params = pltpu.CompilerParams(use_tc_tiling_on_sc=False,
                                  needs_layout_passes=False)


@functools.partial(
    pl.kernel,
    out_type=jax.ShapeDtypeStruct((NC, NP, 16), jnp.float32),
    mesh=_sc_mesh,
    scratch_types=[
        pltpu.VMEM((CH, K), jnp.int32),
        pltpu.VMEM((K, 16), jnp.float32),
        pltpu.VMEM_SHARED((NP, 16), jnp.float32),
        pltpu.SemaphoreType.DMA,
    ],
    compiler_params=_sc_params,
)
def _deg_sc(dstr_hbm, zeros_hbm, ones_hbm, out_hbm, dst_v, ones_v, acc, sem):
    c = lax.axis_index("c")
    s = lax.axis_index("s")
    r0 = s * RPT
    pltpu.sync_copy(zeros_hbm.at[pl.ds(r0, RPT)], acc.at[pl.ds(r0, RPT)])
    pltpu.sync_copy(dstr_hbm.at[pl.ds(s * CH, CH)], dst_v)
    pltpu.sync_copy(ones_hbm, ones_v)
    plsc.subcore_barrier()

    def body(j, carry):
        pltpu.sync_copy(ones_v, acc.at[dst_v.at[j]], add=True)
        return carry

    lax.fori_loop(c * CHD, jnp.minimum(CH, (c + 1) * CHD), body, 0)
    plsc.subcore_barrier()
    pltpu.sync_copy(acc.at[pl.ds(r0, RPT)], out_hbm.at[c, pl.ds(r0, RPT)])


@functools.partial(
    pl.kernel,
    out_type=jax.ShapeDtypeStruct((NC, NP, HH), jnp.float32),
    mesh=_sc_mesh,
    scratch_types=[
        pltpu.VMEM((CH, K), jnp.int32),
        pltpu.VMEM((CH, K), jnp.int32),
        [pltpu.VMEM((K, HH), jnp.bfloat16) for _ in range(NBUF)],
        pltpu.VMEM((K, HH), jnp.float32),
        pltpu.VMEM_SHARED((NP, HH), jnp.float32),
        [pltpu.SemaphoreType.DMA for _ in range(NBUF)],
    ],
    compiler_params=_sc_params,
)
def _scatter_sc(mtb_hbm, srcr_hbm, dstr_hbm, zeros_hbm, out_hbm,
                src_v, dst_v, rows, frow, acc, gsems):
    c = lax.axis_index("c")
    s = lax.axis_index("s")
    r0 = s * RPT

    pltpu.sync_copy(zeros_hbm.at[pl.ds(r0, RPT)], acc.at[pl.ds(r0, RPT)])
    pltpu.sync_copy(srcr_hbm.at[pl.ds(s * CH, CH)], src_v)
    pltpu.sync_copy(dstr_hbm.at[pl.ds(s * CH, CH)], dst_v)
    plsc.subcore_barrier()
    table = mtb_hbm.at[c]

    def convert(src_ref):
        def crow(r, carry):
            for g in range(HH // 32):
                w = plsc.bitcast(src_ref[r, pl.ds(g * 32, 32)], jnp.int32)
                lo = lax.shift_left(w, jnp.full((16,), 16, jnp.int32))
                hi = lax.bitwise_and(w, jnp.full((16,), -65536, jnp.int32))
                frow[r, pl.ds(g * 32, 16)] = plsc.bitcast(lo, jnp.float32)
                frow[r, pl.ds(g * 32 + 16, 16)] = plsc.bitcast(hi, jnp.float32)
            return carry
        lax.fori_loop(0, K, crow, 0)

    pltpu.async_copy(table.at[src_v.at[0]], rows[0], gsems[0])

    def body(i, carry):
        j = 2 * i
        d1 = pltpu.async_copy(table.at[src_v.at[j + 1]], rows[1], gsems[1])
        pltpu.make_async_copy(table.at[src_v.at[j]], rows[0], gsems[0]).wait()
        convert(rows[0])
        pltpu.sync_copy(frow, acc.at[dst_v.at[j]], add=True)

        @pl.when(j + 2 < CH)
        def _():
            pltpu.async_copy(table.at[src_v.at[j + 2]], rows[0], gsems[0])

        d1.wait()
        convert(rows[1])
        pltpu.sync_copy(frow, acc.at[dst_v.at[j + 1]], add=True)
        return carry

    lax.fori_loop(0, CH // 2, body, 0)
    plsc.subcore_barrier()
    pltpu.sync_copy(acc.at[pl.ds(r0, RPT)], out_hbm.at[c, pl.ds(r0, RPT)])


def _write_split(out2_ref, mtn):
    out2_ref[0, :N, :] = mtn[:, :HH]
    out2_ref[1, :N, :] = mtn[:, HH:]
    out2_ref[:, N:, :] = jnp.zeros((NC, NP - N, HH), jnp.float32)


def _read_split(ref2):
    return jnp.concatenate([ref2[0, :N, :], ref2[1, :N, :]], axis=1)


def _pre_body(x_ref, wi_ref, bi_ref, w0_ref, deg2_ref, dinv_ref, mt_ref):
    deg = deg2_ref[0, :N, 0:1] + deg2_ref[1, :N, 0:1] + 1.0
    dinv = lax.rsqrt(deg)
    h = jnp.dot(x_ref[...], wi_ref[...], preferred_element_type=jnp.float32)
    h = h + bi_ref[...]
    m = jnp.dot(h, w0_ref[...], preferred_element_type=jnp.float32)
    dinv_ref[...] = dinv
    _write_split(mt_ref, dinv * m)


_pre_tc = pl.pallas_call(
    _pre_body,
    out_shape=(jax.ShapeDtypeStruct((N, 1), jnp.float32),
               jax.ShapeDtypeStruct((NC, NP, HH), jnp.float32)),
)


def _bn_relu(y, gamma, beta):
    mu = jnp.mean(y, axis=0, keepdims=True)
    var = jnp.mean((y - mu) ** 2, axis=0, keepdims=True)
    return jax.nn.relu((y - mu) * lax.rsqrt(var + EPS) * gamma + beta)


def _layer_body(s_ref, mt_ref, dinv_ref, b_ref, g_ref, be_ref, wn_ref,
                mtn_ref):
    dinv = dinv_ref[...]
    y = dinv * (_read_split(s_ref) + _read_split(mt_ref)) + b_ref[...]
    h = _bn_relu(y, g_ref[...], be_ref[...])
    m = jnp.dot(h, wn_ref[...], preferred_element_type=jnp.float32)
    _write_split(mtn_ref, dinv * m)


_layer_tc = pl.pallas_call(
    _layer_body,
    out_shape=jax.ShapeDtypeStruct((NC, NP, HH), jnp.float32),
)


def _final_body(s_ref, mt_ref, dinv_ref, b_ref, g_ref, be_ref, batch_ref,
                w1_ref, b1_ref, w2_ref, b2_ref, w3_ref, b3_ref, out_ref):
    dinv = dinv_ref[...]
    y = dinv * (_read_split(s_ref) + _read_split(mt_ref)) + b_ref[...]
    h = _bn_relu(y, g_ref[...], be_ref[...])
    iota = lax.broadcasted_iota(jnp.int32, (1, B), 1)
    p = (batch_ref[...] == iota).astype(jnp.float32)
    dn = (((0,), (0,)), ((), ()))
    sums = lax.dot_general(p, h, dimension_numbers=dn,
                           preferred_element_type=jnp.float32)
    cnt = lax.dot_general(p, jnp.ones((N, 1), jnp.float32), dimension_numbers=dn,
                          preferred_element_type=jnp.float32)
    pooled = sums / jnp.maximum(cnt, 1.0)
    o = jax.nn.relu(pooled)
    o = jax.nn.relu(jnp.dot(o, w1_ref[...], preferred_element_type=jnp.float32)
                    + b1_ref[...])
    o = jax.nn.relu(jnp.dot(o, w2_ref[...], preferred_element_type=jnp.float32)
                    + b2_ref[...])
    out_ref[...] = (jnp.dot(o, w3_ref[...], preferred_element_type=jnp.float32)
                    + b3_ref[...])


_final_tc = pl.pallas_call(
    _final_body,
    out_shape=jax.ShapeDtypeStruct((B, C), jnp.float32),
)


def _pad2(a, r, c):
    return jnp.zeros((r, c), a.dtype).at[:a.shape[0], :a.shape[1]].set(a)


def _pad_row(a, c):
    return jnp.zeros((1, c), a.dtype).at[0, :a.shape[0]].set(a)


def _bf16_interleave(mt):
    return (mt.astype(jnp.bfloat16)
              .reshape(NC, NP * HH // 32, 2, 16)
              .swapaxes(2, 3)
              .reshape(NC, NP, HH))


def kernel(x, edge_index, batch, W_in, b_in, Ws, bs, gammas, betas,
           W1, b1, W2, b2, W3, b3):
    src = edge_index[0].astype(jnp.int32)
    dst = edge_index[1].astype(jnp.int32)
    padidx = jnp.full((EPAD - E,), N, jnp.int32)
    srcr = jnp.concatenate([src, padidx]).reshape(NS * CH, K)
    dstr = jnp.concatenate([dst, padidx]).reshape(NS * CH, K)
    zeros = jnp.zeros((NP, HH), jnp.float32)
    zeros16 = jnp.zeros((NP, 16), jnp.float32)
    ones16 = jnp.ones((K, 16), jnp.float32)

    wi_p = _pad2(W_in, D_IN, HP)
    bi_p = _pad_row(b_in, HP)
    ws_p = [_pad2(Ws[i], HP, HP) for i in range(NUM_LAYERS)]
    bs_p = [_pad_row(bs[i], HP) for i in range(NUM_LAYERS)]
    g_p = [_pad_row(gammas[i], HP) for i in range(NUM_LAYERS)]
    be_p = [_pad_row(betas[i], HP) for i in range(NUM_LAYERS)]
    w1_p = _pad2(W1, HP, 128)
    b1_p = _pad_row(b1, 128)
    w2_p = _pad2(W2, 128, 128)
    b2_p = _pad_row(b2, 128)
    w3_p = _pad2(W3, 128, C)
    b3_p = _pad_row(b3, C)
    batch2 = batch.astype(jnp.int32).reshape(N, 1)

    deg2 = _deg_sc(dstr, zeros16, ones16)
    dinv, mt = _pre_tc(x, wi_p, bi_p, ws_p[0], deg2)
    for i in range(NUM_LAYERS - 1):
        s2 = _scatter_sc(_bf16_interleave(mt), srcr, dstr, zeros)
        mt = _layer_tc(s2, mt, dinv, bs_p[i], g_p[i], be_p[i], ws_p[i + 1])
    s2 = _scatter_sc(_bf16_interleave(mt), srcr, dstr, zeros)
    return _final_tc(s2, mt, dinv, bs_p[NUM_LAYERS - 1],
                     g_p[NUM_LAYERS - 1], be_p[NUM_LAYERS - 1], batch2,
                     w1_p, b1_p, w2_p, b2_p, w3_p, b3_p)


# speedup vs baseline: 1.1512x; 1.1512x over previous
import functools

import jax
import jax.numpy as jnp
from jax import lax
from jax.experimental import pallas as pl
from jax.experimental.pallas import tpu as pltpu
from jax.experimental.pallas import tpu_sc as plsc

N = 10000
E = 320000
D_IN = 128
H = 146
NUM_LAYERS = 4
B = 8
C = 10
EPS = 1e-5

HP = 192
HH = HP // 2
NC = 2
NS = 16
K = 128
NBUF = 2
CH = NBUF * (-(-E // (NS * K * NBUF)))
EPAD = NS * CH * K
NP = N + 112
RPT = NP // NS
CHD = -(-CH // 2)

_sc_mesh = plsc.VectorSubcoreMesh(
    core_axis_name="c", subcore_axis_name="s", num_cores=NC, num_subcores=NS)
_sc_params = pltpu.CompilerParams(use_tc_tiling_on_sc=False,
                                  needs_layout_passes=False)


@functools.partial(
    pl.kernel,
    out_type=jax.ShapeDtypeStruct((NC, NP, 16), jnp.float32),
    mesh=_sc_mesh,
    scratch_types=[
        pltpu.VMEM((CH, K), jnp.int32),
        pltpu.VMEM((K, 16), jnp.float32),
        pltpu.VMEM_SHARED((NP, 16), jnp.float32),
        pltpu.SemaphoreType.DMA,
    ],
    compiler_params=_sc_params,
)
def _deg_sc(dstr_hbm, zeros_hbm, ones_hbm, out_hbm, dst_v, ones_v, acc, sem):
    c = lax.axis_index("c")
    s = lax.axis_index("s")
    r0 = s * RPT
    pltpu.sync_copy(zeros_hbm.at[pl.ds(r0, RPT)], acc.at[pl.ds(r0, RPT)])
    pltpu.sync_copy(dstr_hbm.at[pl.ds(s * CH, CH)], dst_v)
    pltpu.sync_copy(ones_hbm, ones_v)
    plsc.subcore_barrier()

    def body(j, carry):
        pltpu.sync_copy(ones_v, acc.at[dst_v.at[j]], add=True)
        return carry

    lax.fori_loop(c * CHD, jnp.minimum(CH, (c + 1) * CHD), body, 0)
    plsc.subcore_barrier()
    pltpu.sync_copy(acc.at[pl.ds(r0, RPT)], out_hbm.at[c, pl.ds(r0, RPT)])


@functools.partial(
    pl.kernel,
    out_type=jax.ShapeDtypeStruct((NC, NP, HH), jnp.float32),
    mesh=_sc_mesh,
    scratch_types=[
        pltpu.VMEM((CH, K), jnp.int32),
        pltpu.VMEM((CH, K), jnp.int32),
        [pltpu.VMEM((K, HH), jnp.bfloat16) for _ in range(NBUF)],
        pltpu.VMEM((K, HH), jnp.float32),
        pltpu.VMEM_SHARED((NP, HH), jnp.float32),
        [pltpu.SemaphoreType.DMA for _ in range(NBUF)],
    ],
    compiler_params=_sc_params,
)
def _scatter_sc(mtb_hbm, srcr_hbm, dstr_hbm, zeros_hbm, out_hbm,
                src_v, dst_v, rows, frow, acc, gsems):
    c = lax.axis_index("c")
    s = lax.axis_index("s")
    r0 = s * RPT

    pltpu.sync_copy(zeros_hbm.at[pl.ds(r0, RPT)], acc.at[pl.ds(r0, RPT)])
    pltpu.sync_copy(srcr_hbm.at[pl.ds(s * CH, CH)], src_v)
    pltpu.sync_copy(dstr_hbm.at[pl.ds(s * CH, CH)], dst_v)
    plsc.subcore_barrier()
    table = mtb_hbm.at[c]

    def convert(src_ref):
        @plsc.parallel_loop(0, K, unroll=8)
        def crow(r):
            for g in range(HH // 32):
                w = plsc.bitcast(src_ref[r, pl.ds(g * 32, 32)], jnp.int32)
                lo = lax.shift_left(w, jnp.full((16,), 16, jnp.int32))
                hi = lax.bitwise_and(w, jnp.full((16,), -65536, jnp.int32))
                frow[r, pl.ds(g * 32, 16)] = plsc.bitcast(lo, jnp.float32)
                frow[r, pl.ds(g * 32 + 16, 16)] = plsc.bitcast(hi, jnp.float32)

    pltpu.async_copy(table.at[src_v.at[0]], rows[0], gsems[0])

    def body(i, carry):
        j = 2 * i
        d1 = pltpu.async_copy(table.at[src_v.at[j + 1]], rows[1], gsems[1])
        pltpu.make_async_copy(table.at[src_v.at[j]], rows[0], gsems[0]).wait()
        convert(rows[0])
        pltpu.sync_copy(frow, acc.at[dst_v.at[j]], add=True)

        @pl.when(j + 2 < CH)
        def _():
            pltpu.async_copy(table.at[src_v.at[j + 2]], rows[0], gsems[0])

        d1.wait()
        convert(rows[1])
        pltpu.sync_copy(frow, acc.at[dst_v.at[j + 1]], add=True)
        return carry

    lax.fori_loop(0, CH // 2, body, 0)
    plsc.subcore_barrier()
    pltpu.sync_copy(acc.at[pl.ds(r0, RPT)], out_hbm.at[c, pl.ds(r0, RPT)])


def _write_split(out2_ref, mtn):
    out2_ref[0, :N, :] = mtn[:, :HH]
    out2_ref[1, :N, :] = mtn[:, HH:]
    out2_ref[:, N:, :] = jnp.zeros((NC, NP - N, HH), jnp.float32)


def _read_split(ref2):
    return jnp.concatenate([ref2[0, :N, :], ref2[1, :N, :]], axis=1)


def _pre_body(x_ref, wi_ref, bi_ref, w0_ref, deg2_ref, dinv_ref, mt_ref):
    deg = deg2_ref[0, :N, 0:1] + deg2_ref[1, :N, 0:1] + 1.0
    dinv = lax.rsqrt(deg)
    h = jnp.dot(x_ref[...], wi_ref[...], preferred_element_type=jnp.float32)
    h = h + bi_ref[...]
    m = jnp.dot(h, w0_ref[...], preferred_element_type=jnp.float32)
    dinv_ref[...] = dinv
    _write_split(mt_ref, dinv * m)


_pre_tc = pl.pallas_call(
    _pre_body,
    out_shape=(jax.ShapeDtypeStruct((N, 1), jnp.float32),
               jax.ShapeDtypeStruct((NC, NP, HH), jnp.float32)),
)


def _bn_relu(y, gamma, beta):
    mu = jnp.mean(y, axis=0, keepdims=True)
    var = jnp.mean((y - mu) ** 2, axis=0, keepdims=True)
    return jax.nn.relu((y - mu) * lax.rsqrt(var + EPS) * gamma + beta)


def _layer_body(s_ref, mt_ref, dinv_ref, b_ref, g_ref, be_ref, wn_ref,
                mtn_ref):
    dinv = dinv_ref[...]
    y = dinv * (_read_split(s_ref) + _read_split(mt_ref)) + b_ref[...]
    h = _bn_relu(y, g_ref[...], be_ref[...])
    m = jnp.dot(h, wn_ref[...], preferred_element_type=jnp.float32)
    _write_split(mtn_ref, dinv * m)


_layer_tc = pl.pallas_call(
    _layer_body,
    out_shape=jax.ShapeDtypeStruct((NC, NP, HH), jnp.float32),
)


def _final_body(s_ref, mt_ref, dinv_ref, b_ref, g_ref, be_ref, batch_ref,
                w1_ref, b1_ref, w2_ref, b2_ref, w3_ref, b3_ref, out_ref):
    dinv = dinv_ref[...]
    y = dinv * (_read_split(s_ref) + _read_split(mt_ref)) + b_ref[...]
    h = _bn_relu(y, g_ref[...], be_ref[...])
    iota = lax.broadcasted_iota(jnp.int32, (1, B), 1)
    p = (batch_ref[...] == iota).astype(jnp.float32)
    dn = (((0,), (0,)), ((), ()))
    sums = lax.dot_general(p, h, dimension_numbers=dn,
                           preferred_element_type=jnp.float32)
    cnt = lax.dot_general(p, jnp.ones((N, 1), jnp.float32), dimension_numbers=dn,
                          preferred_element_type=jnp.float32)
    pooled = sums / jnp.maximum(cnt, 1.0)
    o = jax.nn.relu(pooled)
    o = jax.nn.relu(jnp.dot(o, w1_ref[...], preferred_element_type=jnp.float32)
                    + b1_ref[...])
    o = jax.nn.relu(jnp.dot(o, w2_ref[...], preferred_element_type=jnp.float32)
                    + b2_ref[...])
    out_ref[...] = (jnp.dot(o, w3_ref[...], preferred_element_type=jnp.float32)
                    + b3_ref[...])


_final_tc = pl.pallas_call(
    _final_body,
    out_shape=jax.ShapeDtypeStruct((B, C), jnp.float32),
)


def _pad2(a, r, c):
    return jnp.zeros((r, c), a.dtype).at[:a.shape[0], :a.shape[1]].set(a)


def _pad_row(a, c):
    return jnp.zeros((1, c), a.dtype).at[0, :a.shape[0]].set(a)


def _bf16_interleave(mt):
    return (mt.astype(jnp.bfloat16)
              .reshape(NC, NP * HH // 32, 2, 16)
              .swapaxes(2, 3)
              .reshape(NC, NP, HH))


def kernel(x, edge_index, batch, W_in, b_in, Ws, bs, gammas, betas,
           W1, b1, W2, b2, W3, b3):
    src = edge_index[0].astype(jnp.int32)
    dst = edge_index[1].astype(jnp.int32)
    padidx = jnp.full((EPAD - E,), N, jnp.int32)
    srcr = jnp.concatenate([src, padidx]).reshape(NS * CH, K)
    dstr = jnp.concatenate([dst, padidx]).reshape(NS * CH, K)
    zeros = jnp.zeros((NP, HH), jnp.float32)
    zeros16 = jnp.zeros((NP, 16), jnp.float32)
    ones16 = jnp.ones((K, 16), jnp.float32)

    wi_p = _pad2(W_in, D_IN, HP)
    bi_p = _pad_row(b_in, HP)
    ws_p = [_pad2(Ws[i], HP, HP) for i in range(NUM_LAYERS)]
    bs_p = [_pad_row(bs[i], HP) for i in range(NUM_LAYERS)]
    g_p = [_pad_row(gammas[i], HP) for i in range(NUM_LAYERS)]
    be_p = [_pad_row(betas[i], HP) for i in range(NUM_LAYERS)]
    w1_p = _pad2(W1, HP, 128)
    b1_p = _pad_row(b1, 128)
    w2_p = _pad2(W2, 128, 128)
    b2_p = _pad_row(b2, 128)
    w3_p = _pad2(W3, 128, C)
    b3_p = _pad_row(b3, C)
    batch2 = batch.astype(jnp.int32).reshape(N, 1)

    deg2 = _deg_sc(dstr, zeros16, ones16)
    dinv, mt = _pre_tc(x, wi_p, bi_p, ws_p[0], deg2)
    for i in range(NUM_LAYERS - 1):
        s2 = _scatter_sc(_bf16_interleave(mt), srcr, dstr, zeros)
        mt = _layer_tc(s2, mt, dinv, bs_p[i], g_p[i], be_p[i], ws_p[i + 1])
    s2 = _scatter_sc(_bf16_interleave(mt), srcr, dstr, zeros)
    return _final_tc(s2, mt, dinv, bs_p[NUM_LAYERS - 1],
                     g_p[NUM_LAYERS - 1], be_p[NUM_LAYERS - 1], batch2,
                     w1_p, b1_p, w2_p, b2_p, w3_p, b3_p)


# speedup vs baseline: 4.1258x; 3.5839x over previous
import functools

import jax
import jax.numpy as jnp
from jax import lax
from jax.experimental import pallas as pl
from jax.experimental.pallas import tpu as pltpu
from jax.experimental.pallas import tpu_sc as plsc

N = 10000
E = 320000
D_IN = 128
H = 146
NUM_LAYERS = 4
B = 8
C = 10
EPS = 1e-5

HP = 160
HH = HP // 2
NC = 2
NS = 16
K = 128
NBUF = 2
CH = NBUF * (-(-E // (NS * K * NBUF)))
EPAD = NS * CH * K
NP = N + 112
RPT = NP // NS
CHD = -(-CH // 2)

_sc_mesh = plsc.VectorSubcoreMesh(
    core_axis_name="c", subcore_axis_name="s", num_cores=NC, num_subcores=NS)
_sc_params = pltpu.CompilerParams(use_tc_tiling_on_sc=False)


@functools.partial(
    pl.kernel,
    out_type=jax.ShapeDtypeStruct((NC, NP, 16), jnp.float32),
    mesh=_sc_mesh,
    scratch_types=[
        pltpu.VMEM((CH, K), jnp.int32),
        pltpu.VMEM((K, 16), jnp.float32),
        pltpu.VMEM_SHARED((NP, 16), jnp.float32),
        pltpu.SemaphoreType.DMA,
    ],
    compiler_params=_sc_params,
)
def _deg_sc(dstr_hbm, zeros_hbm, ones_hbm, out_hbm, dst_v, ones_v, acc, sem):
    c = lax.axis_index("c")
    s = lax.axis_index("s")
    r0 = s * RPT
    pltpu.sync_copy(zeros_hbm.at[pl.ds(r0, RPT)], acc.at[pl.ds(r0, RPT)])
    pltpu.sync_copy(dstr_hbm.at[pl.ds(s * CH, CH)], dst_v)
    pltpu.sync_copy(ones_hbm, ones_v)
    plsc.subcore_barrier()

    def body(j, carry):
        pltpu.sync_copy(ones_v, acc.at[dst_v.at[j]], add=True)
        return carry

    lax.fori_loop(c * CHD, jnp.minimum(CH, (c + 1) * CHD), body, 0)
    plsc.subcore_barrier()
    pltpu.sync_copy(acc.at[pl.ds(r0, RPT)], out_hbm.at[c, pl.ds(r0, RPT)])


@functools.partial(
    pl.kernel,
    out_type=jax.ShapeDtypeStruct((NC, NP, HH), jnp.float32),
    mesh=_sc_mesh,
    scratch_types=[
        pltpu.VMEM((CH, K), jnp.int32),
        pltpu.VMEM((CH, K), jnp.int32),
        [pltpu.VMEM((K, HH), jnp.float32) for _ in range(NBUF)],
        pltpu.VMEM_SHARED((NP, HH), jnp.float32),
        [pltpu.SemaphoreType.DMA for _ in range(NBUF)],
    ],
    compiler_params=_sc_params,
)
def _scatter_sc(mt2_hbm, srcr_hbm, dstr_hbm, zeros_hbm, out_hbm,
                src_v, dst_v, rows, acc, gsems):
    c = lax.axis_index("c")
    s = lax.axis_index("s")
    r0 = s * RPT

    pltpu.sync_copy(zeros_hbm.at[pl.ds(r0, RPT)], acc.at[pl.ds(r0, RPT)])
    pltpu.sync_copy(srcr_hbm.at[pl.ds(s * CH, CH)], src_v)
    pltpu.sync_copy(dstr_hbm.at[pl.ds(s * CH, CH)], dst_v)
    plsc.subcore_barrier()
    table = mt2_hbm.at[c]

    pltpu.async_copy(table.at[src_v.at[0]], rows[0], gsems[0])

    def body(i, carry):
        j = 2 * i
        d1 = pltpu.async_copy(table.at[src_v.at[j + 1]], rows[1], gsems[1])
        pltpu.make_async_copy(table.at[src_v.at[j]], rows[0], gsems[0]).wait()
        pltpu.sync_copy(rows[0], acc.at[dst_v.at[j]], add=True)

        @pl.when(j + 2 < CH)
        def _():
            pltpu.async_copy(table.at[src_v.at[j + 2]], rows[0], gsems[0])

        d1.wait()
        pltpu.sync_copy(rows[1], acc.at[dst_v.at[j + 1]], add=True)
        return carry

    lax.fori_loop(0, CH // 2, body, 0)
    plsc.subcore_barrier()
    pltpu.sync_copy(acc.at[pl.ds(r0, RPT)], out_hbm.at[c, pl.ds(r0, RPT)])


def _write_split(out2_ref, mtn):
    out2_ref[0, :N, :] = mtn[:, :HH]
    out2_ref[1, :N, :] = mtn[:, HH:]
    out2_ref[:, N:, :] = jnp.zeros((NC, NP - N, HH), jnp.float32)


def _read_split(ref2):
    return jnp.concatenate([ref2[0, :N, :], ref2[1, :N, :]], axis=1)


def _pre_body(x_ref, wi_ref, bi_ref, w0_ref, deg2_ref, dinv_ref, mt_ref):
    deg = deg2_ref[0, :N, 0:1] + deg2_ref[1, :N, 0:1] + 1.0
    dinv = lax.rsqrt(deg)
    h = jnp.dot(x_ref[...], wi_ref[...], preferred_element_type=jnp.float32)
    h = h + bi_ref[...]
    m = jnp.dot(h, w0_ref[...], preferred_element_type=jnp.float32)
    dinv_ref[...] = dinv
    _write_split(mt_ref, dinv * m)


_pre_tc = pl.pallas_call(
    _pre_body,
    out_shape=(jax.ShapeDtypeStruct((N, 1), jnp.float32),
               jax.ShapeDtypeStruct((NC, NP, HH), jnp.float32)),
)


def _bn_relu(y, gamma, beta):
    mu = jnp.mean(y, axis=0, keepdims=True)
    var = jnp.mean((y - mu) ** 2, axis=0, keepdims=True)
    return jax.nn.relu((y - mu) * lax.rsqrt(var + EPS) * gamma + beta)


def _layer_body(s_ref, mt_ref, dinv_ref, b_ref, g_ref, be_ref, wn_ref,
                mtn_ref):
    dinv = dinv_ref[...]
    y = dinv * (_read_split(s_ref) + _read_split(mt_ref)) + b_ref[...]
    h = _bn_relu(y, g_ref[...], be_ref[...])
    m = jnp.dot(h, wn_ref[...], preferred_element_type=jnp.float32)
    _write_split(mtn_ref, dinv * m)


_layer_tc = pl.pallas_call(
    _layer_body,
    out_shape=jax.ShapeDtypeStruct((NC, NP, HH), jnp.float32),
)


def _final_body(s_ref, mt_ref, dinv_ref, b_ref, g_ref, be_ref, batch_ref,
                w1_ref, b1_ref, w2_ref, b2_ref, w3_ref, b3_ref, out_ref):
    dinv = dinv_ref[...]
    y = dinv * (_read_split(s_ref) + _read_split(mt_ref)) + b_ref[...]
    h = _bn_relu(y, g_ref[...], be_ref[...])
    iota = lax.broadcasted_iota(jnp.int32, (1, B), 1)
    p = (batch_ref[...] == iota).astype(jnp.float32)
    dn = (((0,), (0,)), ((), ()))
    sums = lax.dot_general(p, h, dimension_numbers=dn,
                           preferred_element_type=jnp.float32)
    cnt = lax.dot_general(p, jnp.ones((N, 1), jnp.float32), dimension_numbers=dn,
                          preferred_element_type=jnp.float32)
    pooled = sums / jnp.maximum(cnt, 1.0)
    o = jax.nn.relu(pooled)
    o = jax.nn.relu(jnp.dot(o, w1_ref[...], preferred_element_type=jnp.float32)
                    + b1_ref[...])
    o = jax.nn.relu(jnp.dot(o, w2_ref[...], preferred_element_type=jnp.float32)
                    + b2_ref[...])
    out_ref[...] = (jnp.dot(o, w3_ref[...], preferred_element_type=jnp.float32)
                    + b3_ref[...])


_final_tc = pl.pallas_call(
    _final_body,
    out_shape=jax.ShapeDtypeStruct((B, C), jnp.float32),
)


def _pad2(a, r, c):
    return jnp.zeros((r, c), a.dtype).at[:a.shape[0], :a.shape[1]].set(a)


def _pad_row(a, c):
    return jnp.zeros((1, c), a.dtype).at[0, :a.shape[0]].set(a)


def kernel(x, edge_index, batch, W_in, b_in, Ws, bs, gammas, betas,
           W1, b1, W2, b2, W3, b3):
    src = edge_index[0].astype(jnp.int32)
    dst = edge_index[1].astype(jnp.int32)
    padidx = jnp.full((EPAD - E,), N, jnp.int32)
    srcr = jnp.concatenate([src, padidx]).reshape(NS * CH, K)
    dstr = jnp.concatenate([dst, padidx]).reshape(NS * CH, K)
    zeros = jnp.zeros((NP, HH), jnp.float32)
    zeros16 = jnp.zeros((NP, 16), jnp.float32)
    ones16 = jnp.ones((K, 16), jnp.float32)

    wi_p = _pad2(W_in, D_IN, HP)
    bi_p = _pad_row(b_in, HP)
    ws_p = [_pad2(Ws[i], HP, HP) for i in range(NUM_LAYERS)]
    bs_p = [_pad_row(bs[i], HP) for i in range(NUM_LAYERS)]
    g_p = [_pad_row(gammas[i], HP) for i in range(NUM_LAYERS)]
    be_p = [_pad_row(betas[i], HP) for i in range(NUM_LAYERS)]
    w1_p = _pad2(W1, HP, 128)
    b1_p = _pad_row(b1, 128)
    w2_p = _pad2(W2, 128, 128)
    b2_p = _pad_row(b2, 128)
    w3_p = _pad2(W3, 128, C)
    b3_p = _pad_row(b3, C)
    batch2 = batch.astype(jnp.int32).reshape(N, 1)

    deg2 = _deg_sc(dstr, zeros16, ones16)
    dinv, mt = _pre_tc(x, wi_p, bi_p, ws_p[0], deg2)
    for i in range(NUM_LAYERS - 1):
        s2 = _scatter_sc(mt, srcr, dstr, zeros)
        mt = _layer_tc(s2, mt, dinv, bs_p[i], g_p[i], be_p[i], ws_p[i + 1])
    s2 = _scatter_sc(mt, srcr, dstr, zeros)
    return _final_tc(s2, mt, dinv, bs_p[NUM_LAYERS - 1],
                     g_p[NUM_LAYERS - 1], be_p[NUM_LAYERS - 1], batch2,
                     w1_p, b1_p, w2_p, b2_p, w3_p, b3_p)
